# baseline (speedup 1.0000x reference)
import jax
import jax.numpy as jnp
from jax.experimental import pallas as pl
from jax.experimental.pallas import tpu as pltpu

_SEP_TOKEN_IDX = 5
_SEQ_BLK = 256


def _seg_emb_kernel(tok_ref, x_ref, emb_ref, out_ref, carry_ref):
    @pl.when(pl.program_id(0) == 0)
    def _init():
        carry_ref[...] = jnp.zeros_like(carry_ref)

    tok = tok_ref[...]
    is_sep = (tok == _SEP_TOKEN_IDX).astype(jnp.float32)
    blk, batch = tok.shape
    row = jax.lax.broadcasted_iota(jnp.int32, (blk, blk), 0)
    col = jax.lax.broadcasted_iota(jnp.int32, (blk, blk), 1)
    tri = (col <= row).astype(jnp.float32)
    csum = jnp.dot(tri, is_sep, preferred_element_type=jnp.float32)
    carry = carry_ref[...]
    seg = csum - is_sep + carry
    carry_ref[...] = carry + csum[-1:, :]

    nseg, hid = emb_ref.shape
    seg = jnp.minimum(seg, nseg - 1.0)
    e0 = emb_ref[0, :].reshape(1, hid)
    e1 = emb_ref[1, :].reshape(1, hid)
    e2 = emb_ref[2, :].reshape(1, hid)
    for b in range(batch):
        s = seg[:, b][:, None]
        emb = jnp.where(s == 0, e0, jnp.where(s == 1, e1, e2))
        sl = pl.ds(b * hid, hid)
        out_ref[:, sl] = x_ref[:, sl] + emb


def kernel(x, tokens, emb_table):
    seq, batch, hid = x.shape
    nseg = emb_table.shape[0]
    tokens = tokens.astype(jnp.int32)
    x2 = x.reshape(seq, batch * hid)
    grid = seq // _SEQ_BLK
    out2 = pl.pallas_call(
        _seg_emb_kernel,
        grid=(grid,),
        in_specs=[
            pl.BlockSpec((_SEQ_BLK, batch), lambda i: (i, 0)),
            pl.BlockSpec((_SEQ_BLK, batch * hid), lambda i: (i, 0)),
            pl.BlockSpec((nseg, hid), lambda i: (0, 0)),
        ],
        out_specs=pl.BlockSpec((_SEQ_BLK, batch * hid), lambda i: (i, 0)),
        out_shape=jax.ShapeDtypeStruct((seq, batch * hid), x.dtype),
        scratch_shapes=[pltpu.VMEM((1, batch), jnp.float32)],
    )(tokens, x2, emb_table)
    return out2.reshape(seq, batch, hid)


# speedup vs baseline: 3.9729x; 3.9729x over previous
import jax
import jax.numpy as jnp
from jax.experimental import pallas as pl
from jax.experimental.pallas import tpu as pltpu

_SEP_TOKEN_IDX = 5
_SEQ_BLK = 512


def _seg_emb_kernel(tok_ref, x_ref, emb_ref, out_ref, carry_ref):
    @pl.when(pl.program_id(0) == 0)
    def _init():
        carry_ref[...] = jnp.zeros_like(carry_ref)

    tok = tok_ref[...]
    is_sep = (tok == _SEP_TOKEN_IDX).astype(jnp.float32)
    blk, batch = tok.shape
    row = jax.lax.broadcasted_iota(jnp.int32, (blk, blk), 0)
    col = jax.lax.broadcasted_iota(jnp.int32, (blk, blk), 1)
    tri = (col <= row).astype(jnp.float32)
    csum = jnp.dot(tri, is_sep, preferred_element_type=jnp.float32)
    carry = carry_ref[...]
    seg = csum - is_sep + carry
    carry_ref[...] = carry + csum[-1:, :]

    nseg, hid = emb_ref.shape
    seg = jnp.minimum(seg, nseg - 1.0)
    seg3 = seg[:, :, None]
    e0 = emb_ref[0, :].reshape(1, 1, hid)
    e1 = emb_ref[1, :].reshape(1, 1, hid)
    e2 = emb_ref[2, :].reshape(1, 1, hid)
    emb = jnp.where(seg3 == 0, e0, jnp.where(seg3 == 1, e1, e2))
    out_ref[...] = x_ref[...] + emb


def kernel(x, tokens, emb_table):
    seq, batch, hid = x.shape
    nseg = emb_table.shape[0]
    tokens = tokens.astype(jnp.int32)
    grid = seq // _SEQ_BLK
    return pl.pallas_call(
        _seg_emb_kernel,
        grid=(grid,),
        in_specs=[
            pl.BlockSpec((_SEQ_BLK, batch), lambda i: (i, 0)),
            pl.BlockSpec((_SEQ_BLK, batch, hid), lambda i: (i, 0, 0)),
            pl.BlockSpec((nseg, hid), lambda i: (0, 0)),
        ],
        out_specs=pl.BlockSpec((_SEQ_BLK, batch, hid), lambda i: (i, 0, 0)),
        out_shape=jax.ShapeDtypeStruct(x.shape, x.dtype),
        scratch_shapes=[pltpu.VMEM((1, batch), jnp.float32)],
    )(tokens, x, emb_table)
